# baseline (speedup 1.0000x reference)
import functools

import jax
import jax.numpy as jnp
from jax import lax
from jax.experimental import pallas as pl
from jax.experimental.pallas import tpu as pltpu
from jax.experimental.pallas import tpu_sc as plsc

_IDX_PAD = 304
_NEG_INF = float("-inf")


def _topk_body(conf_ref, mask_ref, topk_ref, idxg_ref, x_ref):
    bs, N, C = conf_ref.shape
    K = N - 600

    x_ref[...] = jnp.max(conf_ref[...], axis=-1)
    ii = lax.broadcasted_iota(jnp.int32, (bs, N), 1)
    ck = lax.broadcasted_iota(jnp.int32, (bs, _IDX_PAD), 1)

    def round_fn(r, _):
        x = x_ref[...]
        mx = jnp.max(x, axis=1, keepdims=True)
        eq = x == mx
        idxc = jnp.min(jnp.where(eq, ii, N), axis=1, keepdims=True)
        sel = ck == r
        topk_ref[...] = jnp.where(sel, mx, topk_ref[...])
        idxg_ref[...] = jnp.where(sel, idxc, idxg_ref[...])
        x_ref[...] = jnp.where(ii == idxc, _NEG_INF, x)
        return 0

    lax.fori_loop(0, K, round_fn, 0, unroll=False)

    m = mask_ref[...] != 0
    raw = idxg_ref[...]
    T = N - K
    blended = jnp.where(m, raw, T + ck)
    boff = lax.broadcasted_iota(jnp.int32, (bs, _IDX_PAD), 0) * N
    idxg_ref[...] = jnp.where(ck < K, blended, ck - K) + boff


def _run_topk(confidence, mask_col):
    bs, N, C = confidence.shape
    return pl.pallas_call(
        _topk_body,
        in_specs=[
            pl.BlockSpec((bs, N, C), lambda: (0, 0, 0)),
            pl.BlockSpec((bs, 1), lambda: (0, 0)),
        ],
        out_specs=(
            pl.BlockSpec((bs, _IDX_PAD), lambda: (0, 0)),
            pl.BlockSpec((bs, _IDX_PAD), lambda: (0, 0)),
        ),
        out_shape=(
            jax.ShapeDtypeStruct((bs, _IDX_PAD), jnp.float32),
            jax.ShapeDtypeStruct((bs, _IDX_PAD), jnp.int32),
        ),
        scratch_shapes=[pltpu.VMEM((bs, N), jnp.float32)],
    )(confidence, mask_col)


def _make_sc_assemble(bs, N, T, D, A):
    K = N - T
    info = plsc.get_sparse_core_info()
    NC, NS = info.num_cores, info.num_subcores
    NW = NC * NS
    per_w = bs // NW
    mesh = plsc.VectorSubcoreMesh(core_axis_name="c", subcore_axis_name="s")

    @functools.partial(
        pl.kernel,
        mesh=mesh,
        out_type=(
            jax.ShapeDtypeStruct((bs, N, D), jnp.float32),
            jax.ShapeDtypeStruct((bs * N * 16,), jnp.float32),
        ),
        scratch_types=[
            pltpu.VMEM((_IDX_PAD,), jnp.int32),
            pltpu.VMEM((256, D), jnp.float32),
            pltpu.VMEM((48, D), jnp.float32),
            pltpu.VMEM((4, D), jnp.float32),
            pltpu.VMEM((N * 16,), jnp.float32),
            pltpu.VMEM((_IDX_PAD * 16,), jnp.float32),
            pltpu.VMEM((600 * 16,), jnp.float32),
            pltpu.VMEM((16,), jnp.int32),
            pltpu.SemaphoreType.DMA,
            pltpu.SemaphoreType.DMA,
            pltpu.SemaphoreType.DMA,
        ],
    )
    def sc_assemble(feat3, feat_flat, cfeat, ancf, cancf, mask_e, idxg,
                    outf, outaf,
                    idx_v, buf_a, buf_e, buf_d, anc_v, out_av, abounce,
                    mask_v, sem_g, sem_c, sem_a):
        wid = lax.axis_index("s") * NC + lax.axis_index("c")
        RA = 16
        for j in range(per_w):
            b = wid * per_w + j
            pltpu.sync_copy(mask_e.at[b], mask_v)
            pltpu.sync_copy(idxg.at[b], idx_v)
            m = mask_v[...][0] != 0

            anc_cp = pltpu.async_copy(
                ancf.at[pl.ds(b * N * RA, N * RA)], anc_v, sem_a)

            @pl.when(m)
            def _():
                pltpu.async_copy(cfeat.at[b], outf.at[b, pl.ds(0, T)], sem_c)
                pltpu.async_copy(cancf.at[pl.ds(b * T * RA, T * RA)],
                                 abounce, sem_c)

            @pl.when(jnp.logical_not(m))
            def _():
                pltpu.async_copy(feat3.at[b, pl.ds(0, T)],
                                 outf.at[b, pl.ds(0, T)], sem_c)
                pltpu.async_copy(ancf.at[pl.ds(b * N * RA, T * RA)],
                                 abounce, sem_c)

            cps = [
                pltpu.async_copy(feat_flat.at[idx_v.at[pl.ds(0, 128)]],
                                 buf_a.at[pl.ds(0, 128)], sem_g),
                pltpu.async_copy(feat_flat.at[idx_v.at[pl.ds(128, 128)]],
                                 buf_a.at[pl.ds(128, 128)], sem_g),
                pltpu.async_copy(feat_flat.at[idx_v.at[pl.ds(256, 48)]],
                                 buf_e, sem_g),
            ]
            anc_cp.wait()
            boff = b * N

            def permute_chunk(c, _):
                idx16 = (idx_v[pl.ds(c * 16, 16)] - boff) * RA
                for l in range(16):
                    s = pl.multiple_of(idx16[l], RA)
                    out_av[pl.ds((c * 16 + l) * RA, RA)] = (
                        anc_v[pl.ds(s, RA)])
                return 0

            lax.fori_loop(0, _IDX_PAD // 16, permute_chunk, 0)
            a_cps = [
                pltpu.async_copy(
                    out_av.at[pl.ds(0, K * RA)],
                    outaf.at[pl.ds(b * N * RA + T * RA, K * RA)], sem_a),
            ]

            for c in cps:
                c.wait()
            out_cps = [
                pltpu.async_copy(buf_a, outf.at[b, pl.ds(T, 256)], sem_g),
                pltpu.async_copy(buf_e.at[pl.ds(0, 40)],
                                 outf.at[b, pl.ds(T + 256, 40)], sem_g),
            ]
            for r in range(4):
                for c16 in range(D // 16):
                    buf_d[r, pl.ds(c16 * 16, 16)] = (
                        buf_e[40 + r, pl.ds(c16 * 16, 16)])
            out_cps.append(
                pltpu.async_copy(buf_d, outf.at[b, pl.ds(T + 296, 4)], sem_g))
            for c in out_cps:
                c.wait()
            for c in a_cps:
                c.wait()
            pltpu.make_async_copy(
                cfeat.at[b], outf.at[b, pl.ds(0, T)], sem_c).wait()
            pltpu.make_async_copy(
                cancf.at[pl.ds(b * T * RA, T * RA)], abounce, sem_c).wait()
            pltpu.sync_copy(abounce, outaf.at[pl.ds(b * N * RA, T * RA)])

    return sc_assemble


def kernel(confidence, instance_feature, anchor, cached_feature,
           cached_anchor, mask):
    bs, N, C = confidence.shape
    D = instance_feature.shape[2]
    A = anchor.shape[2]
    T = cached_feature.shape[1]
    K = N - T

    mask_i32 = mask.astype(jnp.int32)
    topk_p, idxg = _run_topk(confidence, mask_i32[:, None])
    topk = topk_p[:, :K]

    feat_flat = instance_feature.reshape(bs * N, D)
    mask_e = jnp.broadcast_to(mask_i32[:, None], (bs, 16))
    ancf = jnp.pad(anchor, ((0, 0), (0, 0), (0, 16 - A))).reshape(-1)
    cancf = jnp.pad(cached_anchor, ((0, 0), (0, 0), (0, 16 - A))).reshape(-1)
    sc_assemble = _make_sc_assemble(bs, N, T, D, A)
    outf, outaf = sc_assemble(instance_feature, feat_flat, cached_feature,
                              ancf, cancf, mask_e, idxg)
    outa = outaf.reshape(bs, N, 16)[..., :A]
    return outf, outa, topk


# speedup vs baseline: 3.5977x; 3.5977x over previous
import functools

import jax
import jax.numpy as jnp
from jax import lax
from jax.experimental import pallas as pl
from jax.experimental.pallas import tpu as pltpu
from jax.experimental.pallas import tpu_sc as plsc

_IDX_PAD = 304
_NEG_INF = float("-inf")


def _topk_body(conf_ref, mask_ref, topk_ref, idxg_ref, x_ref):
    C, bs, N = conf_ref.shape
    K = N - 600

    x = conf_ref[0]
    for c in range(1, C):
        x = jnp.maximum(x, conf_ref[c])
    x_ref[...] = x
    ii = lax.broadcasted_iota(jnp.int32, (bs, N), 1)
    ck = lax.broadcasted_iota(jnp.int32, (bs, _IDX_PAD), 1)

    def round_fn(r, _):
        xv = x_ref[...]
        mx = jnp.max(xv, axis=1, keepdims=True)
        eq = xv == mx
        idxc = jnp.min(jnp.where(eq, ii, N), axis=1, keepdims=True)
        sel = ck == r
        topk_ref[...] = jnp.where(sel, mx, topk_ref[...])
        idxg_ref[...] = jnp.where(sel, idxc, idxg_ref[...])
        x_ref[...] = jnp.where(ii == idxc, _NEG_INF, xv)
        return 0

    lax.fori_loop(0, K, round_fn, 0, unroll=False)

    m = mask_ref[...] != 0
    mi = mask_ref[...]
    raw = idxg_ref[...]
    T = N - K
    blended = jnp.where(m, raw, T + ck)
    pad = jnp.where(ck == _IDX_PAD - 1, 4 + mi, ck - K)
    boff = lax.broadcasted_iota(jnp.int32, (bs, _IDX_PAD), 0) * N
    idxg_ref[...] = jnp.where(ck < K, blended, pad) + boff


def _run_topk(conf_t, mask_col):
    C, bs, N = conf_t.shape
    return pl.pallas_call(
        _topk_body,
        in_specs=[
            pl.BlockSpec((C, bs, N), lambda: (0, 0, 0)),
            pl.BlockSpec((bs, 1), lambda: (0, 0)),
        ],
        out_specs=(
            pl.BlockSpec((bs, _IDX_PAD), lambda: (0, 0)),
            pl.BlockSpec((bs, _IDX_PAD), lambda: (0, 0)),
        ),
        out_shape=(
            jax.ShapeDtypeStruct((bs, _IDX_PAD), jnp.float32),
            jax.ShapeDtypeStruct((bs, _IDX_PAD), jnp.int32),
        ),
        scratch_shapes=[pltpu.VMEM((bs, N), jnp.float32)],
    )(conf_t, mask_col)


def _make_sc_assemble(bs, N, T, D):
    K = N - T
    info = plsc.get_sparse_core_info()
    NC, NS = info.num_cores, info.num_subcores
    NW = NC * NS
    per_w = bs // NW
    mesh = plsc.VectorSubcoreMesh(core_axis_name="c", subcore_axis_name="s")
    RA = 16

    @functools.partial(
        pl.kernel,
        mesh=mesh,
        out_type=(
            jax.ShapeDtypeStruct((bs, N, D), jnp.float32),
            jax.ShapeDtypeStruct((bs * N * RA,), jnp.float32),
        ),
        scratch_types=[
            pltpu.VMEM((per_w * _IDX_PAD,), jnp.int32),
            pltpu.VMEM((_IDX_PAD, D), jnp.float32),
            pltpu.VMEM((4, D), jnp.float32),
            pltpu.VMEM((N * RA,), jnp.float32),
            pltpu.VMEM((_IDX_PAD * RA,), jnp.float32),
            pltpu.SemaphoreType.DMA,
            pltpu.SemaphoreType.DMA,
        ],
    )
    def sc_assemble(feat_flat, ancf, idxgf, outf, outaf,
                    idx_v, rows_v, buf_d, anc_v, out_av, sem_g, sem_a):
        wid = lax.axis_index("s") * NC + lax.axis_index("c")
        b0 = wid * per_w
        pltpu.sync_copy(
            idxgf.at[pl.ds(b0 * _IDX_PAD, per_w * _IDX_PAD)], idx_v)
        for j in range(per_w):
            b = b0 + j
            ib = j * _IDX_PAD
            boff = b * N

            anc_cp = pltpu.async_copy(
                ancf.at[pl.ds(b * N * RA, N * RA)], anc_v, sem_a)

            cps = [
                pltpu.async_copy(
                    feat_flat.at[idx_v.at[pl.ds(ib, 128)]],
                    rows_v.at[pl.ds(0, 128)], sem_g),
                pltpu.async_copy(
                    feat_flat.at[idx_v.at[pl.ds(ib + 128, 128)]],
                    rows_v.at[pl.ds(128, 128)], sem_g),
                pltpu.async_copy(
                    feat_flat.at[idx_v.at[pl.ds(ib + 256, 48)]],
                    rows_v.at[pl.ds(256, 48)], sem_g),
            ]

            anc_cp.wait()

            def permute_chunk(c, _):
                idx16 = (idx_v[pl.ds(ib + c * 16, 16)] - boff) * RA
                for l in range(16):
                    s = pl.multiple_of(idx16[l], RA)
                    out_av[pl.ds((c * 16 + l) * RA, RA)] = (
                        anc_v[pl.ds(s, RA)])
                return 0

            lax.fori_loop(0, _IDX_PAD // 16, permute_chunk, 0)
            a_cp = pltpu.async_copy(
                out_av.at[pl.ds(0, K * RA)],
                outaf.at[pl.ds(boff * RA + T * RA, K * RA)], sem_a)

            for c in cps:
                c.wait()
            out_cp = pltpu.async_copy(
                rows_v.at[pl.ds(0, 296)], outf.at[b, pl.ds(T, 296)], sem_g)
            for r in range(4):
                for c16 in range(D // 16):
                    buf_d[r, pl.ds(c16 * 16, 16)] = (
                        rows_v[296 + r, pl.ds(c16 * 16, 16)])
            tail_cp = pltpu.async_copy(
                buf_d, outf.at[b, pl.ds(T + 296, 4)], sem_g)
            out_cp.wait()
            tail_cp.wait()
            a_cp.wait()

    return sc_assemble


def _fill_body(mask_ref, cfeat_ref, canc_ref, feat_hbm, ancp_hbm,
               outf_in, outa_in, outf_ref, outa_ref, sem):
    b = pl.program_id(0)
    T = cfeat_ref.shape[1]
    m = mask_ref[b] != 0

    @pl.when(m)
    def _():
        outf_ref[0] = cfeat_ref[0]
        outa_ref[0] = canc_ref[0]

    @pl.when(jnp.logical_not(m))
    def _():
        pltpu.async_copy(feat_hbm.at[b, pl.ds(0, T)],
                         outf_ref.at[0], sem).wait()
        pltpu.async_copy(ancp_hbm.at[b, pl.ds(0, T)],
                         outa_ref.at[0], sem).wait()


def _run_fill(mask_i32, cached_feature, cancp, instance_feature, ancp,
              outf_sc, outa_sc):
    bs, N, D = outf_sc.shape
    T = cached_feature.shape[1]
    return pl.pallas_call(
        _fill_body,
        grid=(bs,),
        in_specs=[
            pl.BlockSpec(memory_space=pltpu.SMEM),
            pl.BlockSpec((1, T, D), lambda b: (b, 0, 0)),
            pl.BlockSpec((1, T, 16), lambda b: (b, 0, 0)),
            pl.BlockSpec(memory_space=pl.ANY),
            pl.BlockSpec(memory_space=pl.ANY),
            pl.BlockSpec(memory_space=pl.ANY),
            pl.BlockSpec(memory_space=pl.ANY),
        ],
        out_specs=(
            pl.BlockSpec((1, T, D), lambda b: (b, 0, 0)),
            pl.BlockSpec((1, T, 16), lambda b: (b, 0, 0)),
        ),
        out_shape=(
            jax.ShapeDtypeStruct((bs, N, D), jnp.float32),
            jax.ShapeDtypeStruct((bs, N, 16), jnp.float32),
        ),
        input_output_aliases={5: 0, 6: 1},
        scratch_shapes=[pltpu.SemaphoreType.DMA],
    )(mask_i32, cached_feature, cancp, instance_feature, ancp,
      outf_sc, outa_sc)


def kernel(confidence, instance_feature, anchor, cached_feature,
           cached_anchor, mask):
    bs, N, C = confidence.shape
    D = instance_feature.shape[2]
    A = anchor.shape[2]
    T = cached_feature.shape[1]
    K = N - T

    mask_i32 = mask.astype(jnp.int32)
    conf_t = jnp.transpose(confidence, (2, 0, 1))
    topk_p, idxg = _run_topk(conf_t, mask_i32[:, None])
    topk = topk_p[:, :K]

    feat_flat = instance_feature.reshape(bs * N, D)
    ancp = jnp.pad(anchor, ((0, 0), (0, 0), (0, 16 - A)))
    cancp = jnp.pad(cached_anchor, ((0, 0), (0, 0), (0, 16 - A)))
    sc_assemble = _make_sc_assemble(bs, N, T, D)
    outf_sc, outaf = sc_assemble(feat_flat, ancp.reshape(-1),
                                 idxg.reshape(-1))
    outf, outa16 = _run_fill(mask_i32, cached_feature, cancp,
                             instance_feature, ancp,
                             outf_sc, outaf.reshape(bs, N, 16))
    return outf, outa16[..., :A], topk


# speedup vs baseline: 3.7846x; 1.0519x over previous
import functools

import jax
import jax.numpy as jnp
from jax import lax
from jax.experimental import pallas as pl
from jax.experimental.pallas import tpu as pltpu
from jax.experimental.pallas import tpu_sc as plsc

_IDX_PAD = 304
_NEG_INF = float("-inf")


def _topk_body(conf_ref, mask_ref, topk_ref, idxg_ref):
    b = pl.program_id(0)
    N = conf_ref.shape[2]
    K = N - 600

    conf = jnp.max(conf_ref[0], axis=0)
    col = conf[:, None]
    row = conf[None, :]
    jj = lax.broadcasted_iota(jnp.int32, (N, N), 0)
    ii = lax.broadcasted_iota(jnp.int32, (N, N), 1)
    gt = (col > row) | ((col == row) & (jj < ii))
    rank = jnp.sum(gt.astype(jnp.int32), axis=0)

    r_iota = lax.broadcasted_iota(jnp.int32, (N, K), 1)
    Eb = rank[:, None] == r_iota
    E = Eb.astype(jnp.float32)
    topk_ref[0, 0] = jnp.sum(E * conf[:, None], axis=0)

    src = lax.broadcasted_iota(jnp.int32, (N, K), 0)
    idx2 = jnp.sum(jnp.where(Eb, src, 0), axis=0, keepdims=True)

    mi = mask_ref[b]
    rk = lax.broadcasted_iota(jnp.int32, (1, K), 1)
    ids2 = jnp.where(mi != 0, idx2, 600 + rk) + b * N
    rkp = lax.broadcasted_iota(jnp.int32, (1, _IDX_PAD - K), 1)
    pad2 = jnp.where(rkp == _IDX_PAD - K - 1, 4 + mi, rkp) + b * N
    idxg_ref[0, 0] = jnp.concatenate([ids2, pad2], axis=1)[0]


def _run_topk(conf_t, mask_i32):
    bs, C, N = conf_t.shape
    K = N - 600
    return pl.pallas_call(
        _topk_body,
        grid=(bs,),
        in_specs=[
            pl.BlockSpec((1, C, N), lambda b: (b, 0, 0)),
            pl.BlockSpec(memory_space=pltpu.SMEM),
        ],
        out_specs=(
            pl.BlockSpec((1, 1, K), lambda b: (b, 0, 0)),
            pl.BlockSpec((1, 1, _IDX_PAD), lambda b: (b, 0, 0)),
        ),
        out_shape=(
            jax.ShapeDtypeStruct((bs, 1, K), jnp.float32),
            jax.ShapeDtypeStruct((bs, 1, _IDX_PAD), jnp.int32),
        ),
    )(conf_t, mask_i32)


def _make_sc_assemble(bs, N, T, D):
    K = N - T
    info = plsc.get_sparse_core_info()
    NC, NS = info.num_cores, info.num_subcores
    NW = NC * NS
    per_w = bs // NW
    mesh = plsc.VectorSubcoreMesh(core_axis_name="c", subcore_axis_name="s")
    RA = 16

    @functools.partial(
        pl.kernel,
        mesh=mesh,
        out_type=(
            jax.ShapeDtypeStruct((bs, N, D), jnp.float32),
            jax.ShapeDtypeStruct((bs * N * RA,), jnp.float32),
        ),
        scratch_types=[
            pltpu.VMEM((per_w * _IDX_PAD,), jnp.int32),
            pltpu.VMEM((_IDX_PAD, D), jnp.float32),
            pltpu.VMEM((4, D), jnp.float32),
            pltpu.VMEM((N * RA,), jnp.float32),
            pltpu.VMEM((_IDX_PAD * RA,), jnp.float32),
            pltpu.SemaphoreType.DMA,
            pltpu.SemaphoreType.DMA,
        ],
    )
    def sc_assemble(feat_flat, ancf, idxgf, outf, outaf,
                    idx_v, rows_v, buf_d, anc_v, out_av, sem_g, sem_a):
        wid = lax.axis_index("s") * NC + lax.axis_index("c")
        b0 = wid * per_w
        pltpu.sync_copy(
            idxgf.at[pl.ds(b0 * _IDX_PAD, per_w * _IDX_PAD)], idx_v)
        for j in range(per_w):
            b = b0 + j
            ib = j * _IDX_PAD
            boff = b * N

            anc_cp = pltpu.async_copy(
                ancf.at[pl.ds(b * N * RA, N * RA)], anc_v, sem_a)

            cps = [
                pltpu.async_copy(
                    feat_flat.at[idx_v.at[pl.ds(ib, 128)]],
                    rows_v.at[pl.ds(0, 128)], sem_g),
                pltpu.async_copy(
                    feat_flat.at[idx_v.at[pl.ds(ib + 128, 128)]],
                    rows_v.at[pl.ds(128, 128)], sem_g),
                pltpu.async_copy(
                    feat_flat.at[idx_v.at[pl.ds(ib + 256, 48)]],
                    rows_v.at[pl.ds(256, 48)], sem_g),
            ]

            anc_cp.wait()

            def permute_chunk(c, _):
                idx16 = (idx_v[pl.ds(ib + c * 16, 16)] - boff) * RA
                for l in range(16):
                    s = pl.multiple_of(idx16[l], RA)
                    out_av[pl.ds((c * 16 + l) * RA, RA)] = (
                        anc_v[pl.ds(s, RA)])
                return 0

            lax.fori_loop(0, _IDX_PAD // 16, permute_chunk, 0)
            a_cp = pltpu.async_copy(
                out_av.at[pl.ds(0, K * RA)],
                outaf.at[pl.ds(boff * RA + T * RA, K * RA)], sem_a)

            for c in cps:
                c.wait()
            out_cp = pltpu.async_copy(
                rows_v.at[pl.ds(0, 296)], outf.at[b, pl.ds(T, 296)], sem_g)
            for r in range(4):
                for c16 in range(D // 16):
                    buf_d[r, pl.ds(c16 * 16, 16)] = (
                        rows_v[296 + r, pl.ds(c16 * 16, 16)])
            tail_cp = pltpu.async_copy(
                buf_d, outf.at[b, pl.ds(T + 296, 4)], sem_g)
            out_cp.wait()
            tail_cp.wait()
            a_cp.wait()

    return sc_assemble


def _fill_body(mask_ref, cfeat_ref, canc_ref, feat_hbm, ancp_hbm,
               outf_in, outa_in, outf_ref, outa_ref, sem):
    b = pl.program_id(0)
    T = cfeat_ref.shape[1]
    m = mask_ref[b] != 0

    @pl.when(m)
    def _():
        outf_ref[0] = cfeat_ref[0]
        outa_ref[0] = canc_ref[0]

    @pl.when(jnp.logical_not(m))
    def _():
        pltpu.async_copy(feat_hbm.at[b, pl.ds(0, T)],
                         outf_ref.at[0], sem).wait()
        pltpu.async_copy(ancp_hbm.at[b, pl.ds(0, T)],
                         outa_ref.at[0], sem).wait()


def _run_fill(mask_i32, cached_feature, cancp, instance_feature, ancp,
              outf_sc, outa_sc):
    bs, N, D = outf_sc.shape
    T = cached_feature.shape[1]
    return pl.pallas_call(
        _fill_body,
        grid=(bs,),
        in_specs=[
            pl.BlockSpec(memory_space=pltpu.SMEM),
            pl.BlockSpec((1, T, D), lambda b: (b, 0, 0)),
            pl.BlockSpec((1, T, 16), lambda b: (b, 0, 0)),
            pl.BlockSpec(memory_space=pl.ANY),
            pl.BlockSpec(memory_space=pl.ANY),
            pl.BlockSpec(memory_space=pl.ANY),
            pl.BlockSpec(memory_space=pl.ANY),
        ],
        out_specs=(
            pl.BlockSpec((1, T, D), lambda b: (b, 0, 0)),
            pl.BlockSpec((1, T, 16), lambda b: (b, 0, 0)),
        ),
        out_shape=(
            jax.ShapeDtypeStruct((bs, N, D), jnp.float32),
            jax.ShapeDtypeStruct((bs, N, 16), jnp.float32),
        ),
        input_output_aliases={5: 0, 6: 1},
        scratch_shapes=[pltpu.SemaphoreType.DMA],
    )(mask_i32, cached_feature, cancp, instance_feature, ancp,
      outf_sc, outa_sc)


def kernel(confidence, instance_feature, anchor, cached_feature,
           cached_anchor, mask):
    bs, N, C = confidence.shape
    D = instance_feature.shape[2]
    A = anchor.shape[2]
    T = cached_feature.shape[1]
    K = N - T

    mask_i32 = mask.astype(jnp.int32)
    conf_t = jnp.transpose(confidence, (0, 2, 1))
    topk_p, idxg = _run_topk(conf_t, mask_i32)
    topk = topk_p.reshape(bs, K)

    feat_flat = instance_feature.reshape(bs * N, D)
    ancp = jnp.pad(anchor, ((0, 0), (0, 0), (0, 16 - A)))
    cancp = jnp.pad(cached_anchor, ((0, 0), (0, 0), (0, 16 - A)))
    sc_assemble = _make_sc_assemble(bs, N, T, D)
    outf_sc, outaf = sc_assemble(feat_flat, ancp.reshape(-1),
                                 idxg.reshape(-1))
    outf, outa16 = _run_fill(mask_i32, cached_feature, cancp,
                             instance_feature, ancp,
                             outf_sc, outaf.reshape(bs, N, 16))
    return outf, outa16[..., :A], topk
